# Initial kernel scaffold; baseline (speedup 1.0000x reference)
import jax
import jax.numpy as jnp
from jax.experimental import pallas as pl


def kernel(audio_semantic, video_semantic, embedding, ema_count, ema_weight, unactivated_count):
    raise NotImplementedError("write your pallas kernel here")


# speedup vs baseline: 1.2398x; 1.2398x over previous
import functools

import jax
import jax.numpy as jnp
import numpy as np
from jax.experimental import pallas as pl

DECAY = 0.99
EPS = 1e-05


def _softmax_stats(dist):
    s = jnp.sqrt(jnp.maximum(dist, 0.0))
    neg = -s
    m1 = jnp.max(neg, axis=-1, keepdims=True)
    e1 = jnp.exp(neg - m1)
    z1 = jnp.sum(e1, axis=-1, keepdims=True)
    ph1 = e1 / z1
    return ph1, e1


def _argmin_idx(dist, iota):
    rmin = jnp.min(dist, axis=-1, keepdims=True)
    big = jnp.int32(dist.shape[-1])
    return jnp.min(jnp.where(dist == rmin, iota, big), axis=-1)


def _main_kernel(a_ref, v_ref, emb_ref,
                 lpa_ref, lpv_ref, p2a_ref, p2v_ref,
                 ca_ref, cv_ref, hwa_ref, hwv_ref, wa_ref, wv_ref,
                 *, T, D, M):
    b = pl.program_id(0)

    @pl.when(b == 0)
    def _():
        hwa_ref[...] = jnp.zeros_like(hwa_ref)
        hwv_ref[...] = jnp.zeros_like(hwv_ref)
        wa_ref[...] = jnp.zeros_like(wa_ref)
        wv_ref[...] = jnp.zeros_like(wv_ref)

    emb = emb_ref[...]
    embsq = jnp.sum(emb * emb, axis=1)[None, :]
    iota = jax.lax.broadcasted_iota(jnp.int32, (T, M), 1)
    log_max_ent = np.float32(np.log(M))

    a = a_ref[0]
    v = v_ref[0]

    results = []
    for x, lp_ref, p2_ref, c_ref in ((a, lpa_ref, p2a_ref, ca_ref),
                                     (v, lpv_ref, p2v_ref, cv_ref)):
        xe = jax.lax.dot_general(x, emb, (((1,), (1,)), ((), ())),
                                 preferred_element_type=jnp.float32)
        xsq = jnp.sum(x * x, axis=1, keepdims=True)
        dist = xsq + embsq - 2.0 * xe
        ph1, e1 = _softmax_stats(dist)
        ent = -jnp.sum(ph1 * jnp.log(ph1 + 1e-05), axis=-1)
        adj = 1.0 - ent / log_max_ent
        lp_ref[0] = jnp.log(ph1 + 1e-10)
        e2 = e1 * e1
        p2_ref[0] = e2 / jnp.sum(e2, axis=-1, keepdims=True)
        idx = _argmin_idx(dist, iota)
        onehot = (iota == idx[:, None]).astype(jnp.float32)
        c_ref[...] = jnp.sum(onehot, axis=0).reshape(1, 1, M)
        results.append((onehot * adj[:, None], adj))

    (pa, _), (pv, _) = results
    hwa_ref[...] += jnp.sum(pa, axis=0)[None, :]
    hwv_ref[...] += jnp.sum(pv, axis=0)[None, :]
    fa = jnp.concatenate([a, v], axis=1)
    fv = jnp.concatenate([v, a], axis=1)
    wa_ref[...] += jax.lax.dot_general(pa, fa, (((0,), (0,)), ((), ())),
                                       preferred_element_type=jnp.float32)
    wv_ref[...] += jax.lax.dot_general(pv, fv, (((0,), (0,)), ((), ())),
                                       preferred_element_type=jnp.float32)


def _scode_kernel(p2a_ref, p2v_ref, lpa_ref, lpv_ref, s1_ref, s2_ref):
    dn = (((2,), (2,)), ((1,), (1,)))
    s1_ref[...] = jax.lax.dot_general(p2a_ref[...], lpv_ref[...], dn,
                                      preferred_element_type=jnp.float32)
    s2_ref[...] = jax.lax.dot_general(p2v_ref[...], lpa_ref[...], dn,
                                      preferred_element_type=jnp.float32)


def _loss_kernel(s1_ref, s2_ref, out_ref, *, T, B):
    eye = (jax.lax.broadcasted_iota(jnp.int32, (B, B), 0) ==
           jax.lax.broadcasted_iota(jnp.int32, (B, B), 1)).astype(jnp.float32)
    losses = []
    for s_ref in (s1_ref, s2_ref):
        sc = s_ref[...]
        mx = jnp.max(-sc)
        es = jnp.exp(sc + mx)
        ssum = jnp.sum(es, axis=-1)
        diag = jnp.sum(es * eye[None, :, :], axis=-1)
        losses.append(-jnp.mean(jnp.log(diag / (ssum + EPS))))
    out_ref[...] = (0.5 * (losses[0] + losses[1])).reshape(1, 1)


def _tail_kernel(hwa_ref, hwv_ref, wa_ref, wv_ref, ca_ref, cv_ref,
                 ecnt_ref, ew_ref, un_ref,
                 emb2_ref, ec2_ref, ew2_ref, unout_ref, eq_ref,
                 *, B, D, M):
    one_m_d = 1.0 - DECAY

    ec = DECAY * ecnt_ref[...] + one_m_d * hwv_ref[...]
    n = jnp.sum(ec)
    ec = (ec + EPS) / (n + M * EPS) * n
    ew = DECAY * ew_ref[...] + 0.5 * one_m_d * (wv_ref[:, :D] + wv_ref[:, D:])

    ec2 = DECAY * ec + one_m_d * hwa_ref[...]
    n2 = jnp.sum(ec2)
    ec2 = (ec2 + EPS) / (n2 + M * EPS) * n2
    ew2 = DECAY * ew + 0.5 * one_m_d * (wa_ref[:, :D] + wa_ref[:, D:])

    ec2_ref[...] = ec2
    ew2_ref[...] = ew2
    emb2_ref[...] = ew2 / ec2.reshape(M, 1)

    ca = ca_ref[:, 0, :]
    cv = cv_ref[:, 0, :]
    total = jnp.sum(ca, axis=0) + jnp.sum(cv, axis=0)
    unout_ref[...] = jnp.where(total[None, :] > 0.0, 0.0, un_ref[...] + 1.0)

    iota = jax.lax.broadcasted_iota(jnp.int32, (B, M), 1)
    big = jnp.int32(M)
    am = jnp.min(jnp.where(ca == jnp.max(ca, axis=-1, keepdims=True), iota, big), axis=-1)
    vm = jnp.min(jnp.where(cv == jnp.max(cv, axis=-1, keepdims=True), iota, big), axis=-1)
    eq_ref[...] = jnp.sum((am == vm).astype(jnp.int32)).reshape(1, 1)


def kernel(audio_semantic, video_semantic, embedding, ema_count, ema_weight,
           unactivated_count):
    B, T, D = audio_semantic.shape
    M = embedding.shape[0]
    f32 = jnp.float32

    nm = functools.partial(jax.ShapeDtypeStruct, dtype=f32)
    main_out = (
        nm((B, T, M)), nm((B, T, M)), nm((B, T, M)), nm((B, T, M)),
        nm((B, 1, M)), nm((B, 1, M)),
        nm((1, M)), nm((1, M)),
        nm((M, 2 * D)), nm((M, 2 * D)),
    )
    tok_spec = pl.BlockSpec((1, T, D), lambda b: (b, 0, 0))
    full_nm_spec = pl.BlockSpec((1, T, M), lambda b: (b, 0, 0))
    cnt_spec = pl.BlockSpec((1, 1, M), lambda b: (b, 0, 0))
    acc1_spec = pl.BlockSpec((1, M), lambda b: (0, 0))
    acc2_spec = pl.BlockSpec((M, 2 * D), lambda b: (0, 0))
    lpa, lpv, p2a, p2v, ca, cv, hwa, hwv, wa, wv = pl.pallas_call(
        functools.partial(_main_kernel, T=T, D=D, M=M),
        grid=(B,),
        in_specs=[tok_spec, tok_spec, pl.BlockSpec((M, D), lambda b: (0, 0))],
        out_specs=(full_nm_spec,) * 4 + (cnt_spec,) * 2 + (acc1_spec,) * 2
                  + (acc2_spec,) * 2,
        out_shape=main_out,
    )(audio_semantic, video_semantic, embedding)

    Tb = 32
    nm_tb_spec = pl.BlockSpec((B, Tb, M), lambda t: (0, t, 0))
    sc_spec = pl.BlockSpec((Tb, B, B), lambda t: (t, 0, 0))
    s1, s2 = pl.pallas_call(
        _scode_kernel,
        grid=(T // Tb,),
        in_specs=[nm_tb_spec] * 4,
        out_specs=(sc_spec, sc_spec),
        out_shape=(nm((T, B, B)), nm((T, B, B))),
    )(p2a, p2v, lpa, lpv)

    loss = pl.pallas_call(
        functools.partial(_loss_kernel, T=T, B=B),
        out_shape=nm((1, 1)),
    )(s1, s2)

    emb2, ec2, ew2, unact, eq = pl.pallas_call(
        functools.partial(_tail_kernel, B=B, D=D, M=M),
        out_shape=(nm((M, D)), nm((1, M)), nm((M, D)), nm((1, M)),
                   jax.ShapeDtypeStruct((1, 1), jnp.int32)),
    )(hwa, hwv, wa, wv, ca, cv, ema_count.reshape(1, M), ema_weight,
      unactivated_count.reshape(1, M))

    return (loss.reshape(()), emb2, ec2.reshape(M), ew2, unact.reshape(M),
            eq.reshape(()))


# speedup vs baseline: 2.1147x; 1.7057x over previous
import functools

import jax
import jax.numpy as jnp
import numpy as np
from jax.experimental import pallas as pl

DECAY = 0.99
EPS = 1e-05


def _softmax_stats(dist):
    s = jnp.sqrt(jnp.maximum(dist, 0.0))
    neg = -s
    m1 = jnp.max(neg, axis=-1, keepdims=True)
    e1 = jnp.exp(neg - m1)
    z1 = jnp.sum(e1, axis=-1, keepdims=True)
    ph1 = e1 / z1
    return ph1, e1


def _argmin_idx(dist, iota):
    rmin = jnp.min(dist, axis=-1, keepdims=True)
    big = jnp.int32(dist.shape[-1])
    return jnp.min(jnp.where(dist == rmin, iota, big), axis=-1)


def _main_kernel(a_ref, v_ref, emb_ref,
                 lpa_ref, lpv_ref, p2a_ref, p2v_ref,
                 ca_ref, cv_ref, hwa_ref, hwv_ref, wa_ref, wv_ref,
                 *, T, D, M):
    b = pl.program_id(0)

    @pl.when(b == 0)
    def _():
        hwa_ref[...] = jnp.zeros_like(hwa_ref)
        hwv_ref[...] = jnp.zeros_like(hwv_ref)
        wa_ref[...] = jnp.zeros_like(wa_ref)
        wv_ref[...] = jnp.zeros_like(wv_ref)

    emb = emb_ref[...]
    embsq = jnp.sum(emb * emb, axis=1)[None, :]
    iota = jax.lax.broadcasted_iota(jnp.int32, (T, M), 1)
    log_max_ent = np.float32(np.log(M))

    a = a_ref[0]
    v = v_ref[0]

    results = []
    for x, lp_ref, p2_ref, c_ref in ((a, lpa_ref, p2a_ref, ca_ref),
                                     (v, lpv_ref, p2v_ref, cv_ref)):
        xe = jax.lax.dot_general(x, emb, (((1,), (1,)), ((), ())),
                                 preferred_element_type=jnp.float32)
        xsq = jnp.sum(x * x, axis=1, keepdims=True)
        dist = xsq + embsq - 2.0 * xe
        ph1, e1 = _softmax_stats(dist)
        ent = -jnp.sum(ph1 * jnp.log(ph1 + 1e-05), axis=-1)
        adj = 1.0 - ent / log_max_ent
        lp_ref[...] = jnp.log(ph1 + 1e-10).reshape(T, 1, 1, M)
        e2 = e1 * e1
        p2_ref[...] = (e2 / jnp.sum(e2, axis=-1, keepdims=True)).reshape(T, 1, 1, M)
        idx = _argmin_idx(dist, iota)
        onehot = (iota == idx[:, None]).astype(jnp.float32)
        c_ref[...] = jnp.sum(onehot, axis=0).reshape(1, 1, M)
        results.append((onehot * adj[:, None], adj))

    (pa, _), (pv, _) = results
    hwa_ref[...] += jnp.sum(pa, axis=0)[None, :]
    hwv_ref[...] += jnp.sum(pv, axis=0)[None, :]
    fa = jnp.concatenate([a, v], axis=1)
    fv = jnp.concatenate([v, a], axis=1)
    wa_ref[...] += jax.lax.dot_general(pa, fa, (((0,), (0,)), ((), ())),
                                       preferred_element_type=jnp.float32)
    wv_ref[...] += jax.lax.dot_general(pv, fv, (((0,), (0,)), ((), ())),
                                       preferred_element_type=jnp.float32)


def _scode_kernel(p2a_ref, p2v_ref, lpa_ref, lpv_ref, s1_ref, s2_ref,
                  *, G, B, M):
    R = G * B
    dn = (((1,), (1,)), ((), ()))
    for a_ref, l_ref, out_ref in ((p2a_ref, lpv_ref, s1_ref),
                                  (p2v_ref, lpa_ref, s2_ref)):
        amat = a_ref[...].reshape(R, M)
        lmat = l_ref[...].reshape(R, M)
        full = jax.lax.dot_general(amat, lmat, dn,
                                   preferred_element_type=jnp.float32)
        out_ref[...] = jnp.stack(
            [full[t * B:(t + 1) * B, t * B:(t + 1) * B] for t in range(G)])


def _loss_kernel(s1_ref, s2_ref, out_ref, *, T, B):
    eye = (jax.lax.broadcasted_iota(jnp.int32, (B, B), 0) ==
           jax.lax.broadcasted_iota(jnp.int32, (B, B), 1)).astype(jnp.float32)
    losses = []
    for s_ref in (s1_ref, s2_ref):
        sc = s_ref[...]
        mx = jnp.max(-sc)
        es = jnp.exp(sc + mx)
        ssum = jnp.sum(es, axis=-1)
        diag = jnp.sum(es * eye[None, :, :], axis=-1)
        losses.append(-jnp.mean(jnp.log(diag / (ssum + EPS))))
    out_ref[...] = (0.5 * (losses[0] + losses[1])).reshape(1, 1)


def _tail_kernel(hwa_ref, hwv_ref, wa_ref, wv_ref, ca_ref, cv_ref,
                 ecnt_ref, ew_ref, un_ref,
                 emb2_ref, ec2_ref, ew2_ref, unout_ref, eq_ref,
                 *, B, D, M):
    one_m_d = 1.0 - DECAY

    ec = DECAY * ecnt_ref[...] + one_m_d * hwv_ref[...]
    n = jnp.sum(ec)
    ec = (ec + EPS) / (n + M * EPS) * n
    ew = DECAY * ew_ref[...] + 0.5 * one_m_d * (wv_ref[:, :D] + wv_ref[:, D:])

    ec2 = DECAY * ec + one_m_d * hwa_ref[...]
    n2 = jnp.sum(ec2)
    ec2 = (ec2 + EPS) / (n2 + M * EPS) * n2
    ew2 = DECAY * ew + 0.5 * one_m_d * (wa_ref[:, :D] + wa_ref[:, D:])

    ec2_ref[...] = ec2
    ew2_ref[...] = ew2
    emb2_ref[...] = ew2 / ec2.reshape(M, 1)

    ca = ca_ref[:, 0, :]
    cv = cv_ref[:, 0, :]
    total = jnp.sum(ca, axis=0) + jnp.sum(cv, axis=0)
    unout_ref[...] = jnp.where(total[None, :] > 0.0, 0.0, un_ref[...] + 1.0)

    iota = jax.lax.broadcasted_iota(jnp.int32, (B, M), 1)
    big = jnp.int32(M)
    am = jnp.min(jnp.where(ca == jnp.max(ca, axis=-1, keepdims=True), iota, big), axis=-1)
    vm = jnp.min(jnp.where(cv == jnp.max(cv, axis=-1, keepdims=True), iota, big), axis=-1)
    eq_ref[...] = jnp.sum((am == vm).astype(jnp.int32)).reshape(1, 1)


def kernel(audio_semantic, video_semantic, embedding, ema_count, ema_weight,
           unactivated_count):
    B, T, D = audio_semantic.shape
    M = embedding.shape[0]
    f32 = jnp.float32

    nm = functools.partial(jax.ShapeDtypeStruct, dtype=f32)
    main_out = (
        nm((T, B, 1, M)), nm((T, B, 1, M)),
        nm((T, B, 1, M)), nm((T, B, 1, M)),
        nm((B, 1, M)), nm((B, 1, M)),
        nm((1, M)), nm((1, M)),
        nm((M, 2 * D)), nm((M, 2 * D)),
    )
    tok_spec = pl.BlockSpec((1, T, D), lambda b: (b, 0, 0))
    full_nm_spec = pl.BlockSpec((T, 1, 1, M), lambda b: (0, b, 0, 0))
    cnt_spec = pl.BlockSpec((1, 1, M), lambda b: (b, 0, 0))
    acc1_spec = pl.BlockSpec((1, M), lambda b: (0, 0))
    acc2_spec = pl.BlockSpec((M, 2 * D), lambda b: (0, 0))
    lpa, lpv, p2a, p2v, ca, cv, hwa, hwv, wa, wv = pl.pallas_call(
        functools.partial(_main_kernel, T=T, D=D, M=M),
        grid=(B,),
        in_specs=[tok_spec, tok_spec, pl.BlockSpec((M, D), lambda b: (0, 0))],
        out_specs=(full_nm_spec,) * 4 + (cnt_spec,) * 2 + (acc1_spec,) * 2
                  + (acc2_spec,) * 2,
        out_shape=main_out,
    )(audio_semantic, video_semantic, embedding)

    G = 128 // B
    nm_tb_spec = pl.BlockSpec((G, B, 1, M), lambda t: (t, 0, 0, 0))
    sc_spec = pl.BlockSpec((G, B, B), lambda t: (t, 0, 0))
    s1, s2 = pl.pallas_call(
        functools.partial(_scode_kernel, G=G, B=B, M=M),
        grid=(T // G,),
        in_specs=[nm_tb_spec] * 4,
        out_specs=(sc_spec, sc_spec),
        out_shape=(nm((T, B, B)), nm((T, B, B))),
    )(p2a, p2v, lpa, lpv)

    loss = pl.pallas_call(
        functools.partial(_loss_kernel, T=T, B=B),
        out_shape=nm((1, 1)),
    )(s1, s2)

    emb2, ec2, ew2, unact, eq = pl.pallas_call(
        functools.partial(_tail_kernel, B=B, D=D, M=M),
        out_shape=(nm((M, D)), nm((1, M)), nm((M, D)), nm((1, M)),
                   jax.ShapeDtypeStruct((1, 1), jnp.int32)),
    )(hwa, hwv, wa, wv, ca, cv, ema_count.reshape(1, M), ema_weight,
      unactivated_count.reshape(1, M))

    return (loss.reshape(()), emb2, ec2.reshape(M), ew2, unact.reshape(M),
            eq.reshape(()))
